# baseline (speedup 1.0000x reference)
import functools

import jax
import jax.numpy as jnp
from jax import lax
from jax.experimental import pallas as pl
from jax.experimental.pallas import tpu as pltpu
from jax.experimental.pallas import tpu_sc as plsc

_NC = 2
_NS = 16
_NW = _NC * _NS

_D = 128
_EB = 3200
_SC_CHUNK = 80


def _node_matmul(x, wT, interpret=False):
    n, d = x.shape

    def body(x_ref, w_ref, o_ref):
        o_ref[...] = jnp.dot(x_ref[...], w_ref[...],
                             preferred_element_type=jnp.float32)

    return pl.pallas_call(
        body,
        out_shape=jax.ShapeDtypeStruct((n, wT.shape[1]), jnp.float32),
        interpret=interpret,
    )(x, wT)


_BIG = 768
_NSUB = _BIG // 128


def _sc_gather_rows(table, idx):
    n, d = table.shape
    e = idx.shape[0]
    per_w = e // _NW
    steps = per_w // _BIG
    rem = per_w - steps * _BIG
    n128 = rem // 128
    tail = rem - n128 * 128
    mesh = plsc.VectorSubcoreMesh(core_axis_name="c", subcore_axis_name="s")

    @functools.partial(
        pl.kernel,
        out_type=jax.ShapeDtypeStruct((e, d), jnp.float32),
        mesh=mesh,
        scratch_types=[
            pltpu.VMEM((_BIG,), jnp.int32),
            pltpu.VMEM((128,), jnp.int32),
            pltpu.VMEM((_BIG, d), jnp.float32),
            pltpu.SemaphoreType.DMA,
        ],
    )
    def k(table_hbm, idx_hbm, out_hbm, idx_v, idxt_v, rows_v, sem):
        cid = lax.axis_index("c")
        sid = lax.axis_index("s")
        wid = sid * _NC + cid
        base0 = wid * per_w

        @pl.loop(0, steps)
        def _(i):
            base = base0 + i * _BIG
            pltpu.sync_copy(idx_hbm.at[pl.ds(base, _BIG)], idx_v)
            hs = [
                pltpu.async_copy(
                    table_hbm.at[idx_v.at[pl.ds(128 * j, 128)]],
                    rows_v.at[pl.ds(128 * j, 128)], sem)
                for j in range(_NSUB)
            ]
            for h in hs:
                h.wait()
            pltpu.sync_copy(rows_v, out_hbm.at[pl.ds(base, _BIG)])

        rbase = base0 + steps * _BIG
        rem_sizes = [128] * n128 + ([tail] if tail else [])
        off = 0
        for sz in rem_sizes:
            pltpu.sync_copy(idx_hbm.at[pl.ds(rbase + off, sz)],
                            idxt_v.at[pl.ds(0, sz)])
            pltpu.async_copy(table_hbm.at[idxt_v.at[pl.ds(0, sz)]],
                             rows_v.at[pl.ds(0, sz)], sem).wait()
            pltpu.sync_copy(rows_v.at[pl.ds(0, sz)],
                            out_hbm.at[pl.ds(rbase + off, sz)])
            off += sz

    return k(table, idx)


def _sc_scatter_add(vals, col, n, zeros_nd):
    e, d = vals.shape
    big = 256
    nsub = big // 128
    per_w = e // _NW
    steps = per_w // big
    erem = per_w - steps * big
    en128 = erem // 128
    etail = erem - en128 * 128
    rpt = (n // _NS) // 8 * 8
    tail = n - 15 * rpt
    mesh = plsc.VectorSubcoreMesh(core_axis_name="c", subcore_axis_name="s")

    @functools.partial(
        pl.kernel,
        out_type=(jax.ShapeDtypeStruct((_NC, n, d), jnp.float32),
                  jax.ShapeDtypeStruct((_NC * n,), jnp.float32)),
        mesh=mesh,
        scratch_types=(
            [pltpu.VMEM((128,), jnp.int32) for _ in range(nsub)]
            + [
                pltpu.VMEM((max(etail, 8),), jnp.int32),
                pltpu.VMEM((big, d), jnp.float32),
                pltpu.VMEM((128,), jnp.float32),
                pltpu.VMEM((tail,), jnp.float32),
                pltpu.VMEM_SHARED((n, d), jnp.float32),
                pltpu.VMEM_SHARED((n,), jnp.float32),
                pltpu.SemaphoreType.DMA,
            ]
        ),
    )
    def k(vals_hbm, col_hbm, znd_hbm, sums_hbm, cnts_hbm,
          i0, i1, idxt_v, vals_v, ones_v, cbuf_v,
          acc_s, cnt_s, sem):
        idxb = [i0, i1]
        cid = lax.axis_index("c")
        sid = lax.axis_index("s")
        wid = sid * _NC + cid
        r0 = sid * rpt

        for i in range(tail // 16):
            cbuf_v[pl.ds(16 * i, 16)] = jnp.zeros((16,), jnp.float32)
        for i in range(128 // 16):
            ones_v[pl.ds(16 * i, 16)] = jnp.full((16,), 1.0, jnp.float32)

        @pl.when(sid < _NS - 1)
        def _():
            pltpu.sync_copy(znd_hbm.at[pl.ds(r0, rpt)],
                            acc_s.at[pl.ds(r0, rpt)])
            pltpu.sync_copy(cbuf_v.at[pl.ds(0, rpt)],
                            cnt_s.at[pl.ds(r0, rpt)])

        @pl.when(sid == _NS - 1)
        def _():
            pltpu.sync_copy(znd_hbm.at[pl.ds(15 * rpt, tail)],
                            acc_s.at[pl.ds(15 * rpt, tail)])
            pltpu.sync_copy(cbuf_v, cnt_s.at[pl.ds(15 * rpt, tail)])

        plsc.subcore_barrier()

        base0 = wid * per_w

        @pl.loop(0, steps)
        def _(i):
            base = base0 + i * big
            hs = [
                pltpu.async_copy(col_hbm.at[pl.ds(base + 128 * j, 128)],
                                 idxb[j], sem)
                for j in range(nsub)
            ]
            hs.append(pltpu.async_copy(vals_hbm.at[pl.ds(base, big)],
                                       vals_v, sem))
            for h in hs:
                h.wait()
            for j in range(nsub):
                pltpu.sync_copy(vals_v.at[pl.ds(128 * j, 128)],
                                acc_s.at[idxb[j]], add=True)
                pltpu.sync_copy(ones_v, cnt_s.at[idxb[j]], add=True)

        rbase = base0 + steps * big
        rem_sizes = [128] * en128 + ([etail] if etail else [])
        off = 0
        for sz, ib in zip(rem_sizes, [i0, i1] * 2):
            if sz == 128:
                pltpu.sync_copy(col_hbm.at[pl.ds(rbase + off, sz)], ib)
                idx_ref = ib
            else:
                pltpu.sync_copy(col_hbm.at[pl.ds(rbase + off, sz)], idxt_v)
                idx_ref = idxt_v
            pltpu.sync_copy(vals_hbm.at[pl.ds(rbase + off, sz)],
                            vals_v.at[pl.ds(0, sz)])
            pltpu.sync_copy(vals_v.at[pl.ds(0, sz)], acc_s.at[idx_ref],
                            add=True)
            pltpu.sync_copy(ones_v.at[pl.ds(0, sz)], cnt_s.at[idx_ref],
                            add=True)
            off += sz

        plsc.subcore_barrier()

        @pl.when(sid < _NS - 1)
        def _():
            pltpu.sync_copy(acc_s.at[pl.ds(r0, rpt)],
                            sums_hbm.at[cid, pl.ds(r0, rpt)])
            pltpu.sync_copy(cnt_s.at[pl.ds(r0, rpt)],
                            cbuf_v.at[pl.ds(0, rpt)])
            pltpu.sync_copy(cbuf_v.at[pl.ds(0, rpt)],
                            cnts_hbm.at[pl.ds(cid * n + r0, rpt)])

        @pl.when(sid == _NS - 1)
        def _():
            pltpu.sync_copy(acc_s.at[pl.ds(15 * rpt, tail)],
                            sums_hbm.at[cid, pl.ds(15 * rpt, tail)])
            pltpu.sync_copy(cnt_s.at[pl.ds(15 * rpt, tail)], cbuf_v)
            pltpu.sync_copy(cbuf_v,
                            cnts_hbm.at[pl.ds(cid * n + 15 * rpt, tail)])

    return k(vals, col, zeros_nd)


def _edge_mlp(g, ea, w1bT, b1, interpret=False):
    e, d = ea.shape
    steps = e // _EB

    def body(g_ref, ea_ref, w_ref, b_ref, pre_ref, st_ref, s1_ref, s2_ref):
        i = pl.program_id(0)

        @pl.when(i == 0)
        def _():
            s1_ref[...] = jnp.zeros_like(s1_ref)
            s2_ref[...] = jnp.zeros_like(s2_ref)

        p = g_ref[...] + jnp.dot(ea_ref[...], w_ref[...],
                                 preferred_element_type=jnp.float32) + b_ref[...]
        pre_ref[...] = p
        s1_ref[...] += jnp.sum(p, axis=0, keepdims=True)
        s2_ref[...] += jnp.sum(p * p, axis=0, keepdims=True)
        st_ref[0:1, :] = s1_ref[...]
        st_ref[1:2, :] = s2_ref[...]

    return pl.pallas_call(
        body,
        grid=(steps,),
        in_specs=[
            pl.BlockSpec((_EB, d), lambda i: (i, 0)),
            pl.BlockSpec((_EB, d), lambda i: (i, 0)),
            pl.BlockSpec((d, d), lambda i: (0, 0)),
            pl.BlockSpec((1, d), lambda i: (0, 0)),
        ],
        out_specs=[
            pl.BlockSpec((_EB, d), lambda i: (i, 0)),
            pl.BlockSpec((8, d), lambda i: (0, 0)),
        ],
        out_shape=[
            jax.ShapeDtypeStruct((e, d), jnp.float32),
            jax.ShapeDtypeStruct((8, d), jnp.float32),
        ],
        scratch_shapes=[
            pltpu.VMEM((1, d), jnp.float32),
            pltpu.VMEM((1, d), jnp.float32),
        ],
        interpret=interpret,
    )(g, ea, w1bT, b1)


def _bn_silu_edges(pre, stats0, stats1, e_total, g1, be1, interpret=False):
    e, d = pre.shape
    steps = e // _EB
    inv_e = 1.0 / e_total

    def body(pre_ref, st0_ref, st1_ref, g_ref, b_ref, s_ref):
        st = st0_ref[...] + st1_ref[...]
        mu = st[0:1, :] * inv_e
        var = st[1:2, :] * inv_e - mu * mu
        rstd = lax.rsqrt(var + 1e-5)
        w = g_ref[...] * rstd
        cc = b_ref[...] - mu * w
        h = pre_ref[...] * w + cc
        s_ref[...] = h * jax.nn.sigmoid(h)

    return pl.pallas_call(
        body,
        grid=(steps,),
        in_specs=[
            pl.BlockSpec((_EB, d), lambda i: (i, 0)),
            pl.BlockSpec((8, d), lambda i: (0, 0)),
            pl.BlockSpec((8, d), lambda i: (0, 0)),
            pl.BlockSpec((1, d), lambda i: (0, 0)),
            pl.BlockSpec((1, d), lambda i: (0, 0)),
        ],
        out_specs=pl.BlockSpec((_EB, d), lambda i: (i, 0)),
        out_shape=jax.ShapeDtypeStruct((e, d), jnp.float32),
        interpret=interpret,
    )(pre, stats0, stats1, g1, be1)


def _node_mlp(x, sums0, sums1, cnts, w2aT, w2bT, b2, g2, be2,
              interpret=False):
    n, d = x.shape
    inv_n = 1.0 / n

    def body(x_ref, su_ref, sv_ref, cn_ref, wa_ref, wb_ref, b_ref, g_ref,
             be_ref, o_ref):
        s = (su_ref[0] + su_ref[1]) + (sv_ref[0] + sv_ref[1])
        cnt = (cn_ref[:, 0:1] + cn_ref[:, 1:2]
               + cn_ref[:, 2:3] + cn_ref[:, 3:4])
        agg = s / jnp.maximum(cnt, 1.0)
        p = (jnp.dot(x_ref[...], wa_ref[...],
                     preferred_element_type=jnp.float32)
             + jnp.dot(agg, wb_ref[...], preferred_element_type=jnp.float32)
             + b_ref[...])
        mu = jnp.sum(p, axis=0, keepdims=True) * inv_n
        var = jnp.sum(p * p, axis=0, keepdims=True) * inv_n - mu * mu
        rstd = lax.rsqrt(var + 1e-5)
        h = (p - mu) * rstd * g_ref[...] + be_ref[...]
        o_ref[...] = h * jax.nn.sigmoid(h)

    return pl.pallas_call(
        body,
        out_shape=jax.ShapeDtypeStruct((n, d), jnp.float32),
        interpret=interpret,
    )(x, sums0, sums1, cnts, w2aT, w2bT, b2, g2, be2)


def kernel(x, edge_index, edge_attr, u, batch, W1, b1, g1, be1, W2, b2, g2,
           be2):
    del u, batch
    n, d = x.shape
    e = edge_index.shape[1]
    row = edge_index[0]
    col = edge_index[1]
    w1aT = jnp.transpose(W1[:, :d])
    w1bT = jnp.transpose(W1[:, d:])
    w2aT = jnp.transpose(W2[:, :d])
    w2bT = jnp.transpose(W2[:, d:])
    b1r = b1.reshape(1, d)
    g1r = g1.reshape(1, d)
    be1r = be1.reshape(1, d)
    b2r = b2.reshape(1, d)
    g2r = g2.reshape(1, d)
    be2r = be2.reshape(1, d)
    zeros_nd = jnp.zeros((n, d), jnp.float32)

    e2 = e // 2
    a = _node_matmul(x, w1aT)
    ga0 = _sc_gather_rows(a, row[:e2])
    ga1 = _sc_gather_rows(a, row[e2:])
    pre0, st0 = _edge_mlp(ga0, edge_attr[:e2], w1bT, b1r)
    pre1, st1 = _edge_mlp(ga1, edge_attr[e2:], w1bT, b1r)
    s0 = _bn_silu_edges(pre0, st0, st1, e, g1r, be1r)
    s1 = _bn_silu_edges(pre1, st0, st1, e, g1r, be1r)
    sums0, cnts0 = _sc_scatter_add(s0, col[:e2], n, zeros_nd)
    sums1, cnts1 = _sc_scatter_add(s1, col[e2:], n, zeros_nd)
    cntT = jnp.concatenate([jnp.transpose(cnts0.reshape(_NC, n)),
                            jnp.transpose(cnts1.reshape(_NC, n))], axis=1)
    out = _node_mlp(x, sums0, sums1, cntT, w2aT, w2bT, b2r, g2r, be2r)
    return out


# speedup vs baseline: 1.1529x; 1.1529x over previous
import functools

import jax
import jax.numpy as jnp
from jax import lax
from jax.experimental import pallas as pl
from jax.experimental.pallas import tpu as pltpu
from jax.experimental.pallas import tpu_sc as plsc

_NC = 2
_NS = 16
_NW = _NC * _NS

_D = 128
_EB = 3200
_SC_CHUNK = 80


def _node_matmul(x, wT, interpret=False):
    n, d = x.shape

    def body(x_ref, w_ref, o_ref):
        o_ref[...] = jnp.dot(x_ref[...], w_ref[...],
                             preferred_element_type=jnp.float32)

    return pl.pallas_call(
        body,
        out_shape=jax.ShapeDtypeStruct((n, wT.shape[1]), jnp.float32),
        interpret=interpret,
    )(x, wT)


_BIG = 768
_NSUB = _BIG // 128


def _sc_gather_rows(table, idx):
    n, d = table.shape
    e = idx.shape[0]
    per_w = e // _NW
    steps = per_w // _BIG
    rem = per_w - steps * _BIG
    n128 = rem // 128
    tail = rem - n128 * 128
    mesh = plsc.VectorSubcoreMesh(core_axis_name="c", subcore_axis_name="s")

    @functools.partial(
        pl.kernel,
        out_type=jax.ShapeDtypeStruct((e, d), jnp.float32),
        mesh=mesh,
        scratch_types=[
            pltpu.VMEM((_BIG,), jnp.int32),
            pltpu.VMEM((128,), jnp.int32),
            pltpu.VMEM((_BIG, d), jnp.float32),
            pltpu.SemaphoreType.DMA,
        ],
    )
    def k(table_hbm, idx_hbm, out_hbm, idx_v, idxt_v, rows_v, sem):
        cid = lax.axis_index("c")
        sid = lax.axis_index("s")
        wid = sid * _NC + cid
        base0 = wid * per_w

        @pl.loop(0, steps)
        def _(i):
            base = base0 + i * _BIG
            pltpu.sync_copy(idx_hbm.at[pl.ds(base, _BIG)], idx_v)
            hs = [
                pltpu.async_copy(
                    table_hbm.at[idx_v.at[pl.ds(128 * j, 128)]],
                    rows_v.at[pl.ds(128 * j, 128)], sem)
                for j in range(_NSUB)
            ]
            for h in hs:
                h.wait()
            pltpu.sync_copy(rows_v, out_hbm.at[pl.ds(base, _BIG)])

        rbase = base0 + steps * _BIG
        rem_sizes = [128] * n128 + ([tail] if tail else [])
        off = 0
        for sz in rem_sizes:
            pltpu.sync_copy(idx_hbm.at[pl.ds(rbase + off, sz)],
                            idxt_v.at[pl.ds(0, sz)])
            pltpu.async_copy(table_hbm.at[idxt_v.at[pl.ds(0, sz)]],
                             rows_v.at[pl.ds(0, sz)], sem).wait()
            pltpu.sync_copy(rows_v.at[pl.ds(0, sz)],
                            out_hbm.at[pl.ds(rbase + off, sz)])
            off += sz

    return k(table, idx)


def _sc_scatter_add(vals, col, n, zeros_nd):
    e, d = vals.shape
    big = 256
    nsub = big // 128
    per_w = e // _NW
    steps = per_w // big
    erem = per_w - steps * big
    en128 = erem // 128
    etail = erem - en128 * 128
    rpt = (n // _NS) // 8 * 8
    tail = n - 15 * rpt
    mesh = plsc.VectorSubcoreMesh(core_axis_name="c", subcore_axis_name="s")

    @functools.partial(
        pl.kernel,
        out_type=(jax.ShapeDtypeStruct((_NC, n, d), jnp.float32),
                  jax.ShapeDtypeStruct((_NC * n,), jnp.float32)),
        mesh=mesh,
        scratch_types=(
            [pltpu.VMEM((128,), jnp.int32) for _ in range(nsub)]
            + [
                pltpu.VMEM((max(etail, 8),), jnp.int32),
                pltpu.VMEM((big, d), jnp.float32),
                pltpu.VMEM((128,), jnp.float32),
                pltpu.VMEM((tail,), jnp.float32),
                pltpu.VMEM_SHARED((n, d), jnp.float32),
                pltpu.VMEM_SHARED((n,), jnp.float32),
                pltpu.SemaphoreType.DMA,
            ]
        ),
    )
    def k(vals_hbm, col_hbm, znd_hbm, sums_hbm, cnts_hbm,
          i0, i1, idxt_v, vals_v, ones_v, cbuf_v,
          acc_s, cnt_s, sem):
        idxb = [i0, i1]
        cid = lax.axis_index("c")
        sid = lax.axis_index("s")
        wid = sid * _NC + cid
        r0 = sid * rpt

        for i in range(tail // 16):
            cbuf_v[pl.ds(16 * i, 16)] = jnp.zeros((16,), jnp.float32)
        for i in range(128 // 16):
            ones_v[pl.ds(16 * i, 16)] = jnp.full((16,), 1.0, jnp.float32)

        @pl.when(sid < _NS - 1)
        def _():
            pltpu.sync_copy(znd_hbm.at[pl.ds(r0, rpt)],
                            acc_s.at[pl.ds(r0, rpt)])
            pltpu.sync_copy(cbuf_v.at[pl.ds(0, rpt)],
                            cnt_s.at[pl.ds(r0, rpt)])

        @pl.when(sid == _NS - 1)
        def _():
            pltpu.sync_copy(znd_hbm.at[pl.ds(15 * rpt, tail)],
                            acc_s.at[pl.ds(15 * rpt, tail)])
            pltpu.sync_copy(cbuf_v, cnt_s.at[pl.ds(15 * rpt, tail)])

        plsc.subcore_barrier()

        base0 = wid * per_w

        @pl.loop(0, steps)
        def _(i):
            base = base0 + i * big
            hs = [
                pltpu.async_copy(col_hbm.at[pl.ds(base + 128 * j, 128)],
                                 idxb[j], sem)
                for j in range(nsub)
            ]
            hs.append(pltpu.async_copy(vals_hbm.at[pl.ds(base, big)],
                                       vals_v, sem))
            for h in hs:
                h.wait()
            for j in range(nsub):
                pltpu.sync_copy(vals_v.at[pl.ds(128 * j, 128)],
                                acc_s.at[idxb[j]], add=True)
                pltpu.sync_copy(ones_v, cnt_s.at[idxb[j]], add=True)

        rbase = base0 + steps * big
        rem_sizes = [128] * en128 + ([etail] if etail else [])
        off = 0
        for sz, ib in zip(rem_sizes, [i0, i1] * 2):
            if sz == 128:
                pltpu.sync_copy(col_hbm.at[pl.ds(rbase + off, sz)], ib)
                idx_ref = ib
            else:
                pltpu.sync_copy(col_hbm.at[pl.ds(rbase + off, sz)], idxt_v)
                idx_ref = idxt_v
            pltpu.sync_copy(vals_hbm.at[pl.ds(rbase + off, sz)],
                            vals_v.at[pl.ds(0, sz)])
            pltpu.sync_copy(vals_v.at[pl.ds(0, sz)], acc_s.at[idx_ref],
                            add=True)
            pltpu.sync_copy(ones_v.at[pl.ds(0, sz)], cnt_s.at[idx_ref],
                            add=True)
            off += sz

        plsc.subcore_barrier()

        @pl.when(sid < _NS - 1)
        def _():
            pltpu.sync_copy(acc_s.at[pl.ds(r0, rpt)],
                            sums_hbm.at[cid, pl.ds(r0, rpt)])
            pltpu.sync_copy(cnt_s.at[pl.ds(r0, rpt)],
                            cbuf_v.at[pl.ds(0, rpt)])
            pltpu.sync_copy(cbuf_v.at[pl.ds(0, rpt)],
                            cnts_hbm.at[pl.ds(cid * n + r0, rpt)])

        @pl.when(sid == _NS - 1)
        def _():
            pltpu.sync_copy(acc_s.at[pl.ds(15 * rpt, tail)],
                            sums_hbm.at[cid, pl.ds(15 * rpt, tail)])
            pltpu.sync_copy(cnt_s.at[pl.ds(15 * rpt, tail)], cbuf_v)
            pltpu.sync_copy(cbuf_v,
                            cnts_hbm.at[pl.ds(cid * n + 15 * rpt, tail)])

    return k(vals, col, zeros_nd)


def _edge_mlp(g, ea, w1bT, b1, interpret=False):
    e, d = ea.shape
    steps = e // _EB

    def body(g_ref, ea_ref, w_ref, b_ref, pre_ref, st_ref, s1_ref, s2_ref):
        i = pl.program_id(0)

        @pl.when(i == 0)
        def _():
            s1_ref[...] = jnp.zeros_like(s1_ref)
            s2_ref[...] = jnp.zeros_like(s2_ref)

        p = g_ref[...] + jnp.dot(ea_ref[...], w_ref[...],
                                 preferred_element_type=jnp.float32) + b_ref[...]
        pre_ref[...] = p.astype(jnp.bfloat16)
        s1_ref[...] += jnp.sum(p, axis=0, keepdims=True)
        s2_ref[...] += jnp.sum(p * p, axis=0, keepdims=True)
        st_ref[0:1, :] = s1_ref[...]
        st_ref[1:2, :] = s2_ref[...]

    return pl.pallas_call(
        body,
        grid=(steps,),
        in_specs=[
            pl.BlockSpec((_EB, d), lambda i: (i, 0)),
            pl.BlockSpec((_EB, d), lambda i: (i, 0)),
            pl.BlockSpec((d, d), lambda i: (0, 0)),
            pl.BlockSpec((1, d), lambda i: (0, 0)),
        ],
        out_specs=[
            pl.BlockSpec((_EB, d), lambda i: (i, 0)),
            pl.BlockSpec((8, d), lambda i: (0, 0)),
        ],
        out_shape=[
            jax.ShapeDtypeStruct((e, d), jnp.bfloat16),
            jax.ShapeDtypeStruct((8, d), jnp.float32),
        ],
        scratch_shapes=[
            pltpu.VMEM((1, d), jnp.float32),
            pltpu.VMEM((1, d), jnp.float32),
        ],
        interpret=interpret,
    )(g, ea, w1bT, b1)


def _bn_silu_edges(pre, stats, e_total, g1, be1, interpret=False):
    e, d = pre.shape
    steps = e // _EB
    inv_e = 1.0 / e_total

    def body(pre_ref, st_ref, g_ref, b_ref, s_ref):
        st = st_ref[...]
        mu = st[0:1, :] * inv_e
        var = st[1:2, :] * inv_e - mu * mu
        rstd = lax.rsqrt(var + 1e-5)
        w = g_ref[...] * rstd
        cc = b_ref[...] - mu * w
        h = pre_ref[...].astype(jnp.float32) * w + cc
        s_ref[...] = h * jax.nn.sigmoid(h)

    return pl.pallas_call(
        body,
        grid=(steps,),
        in_specs=[
            pl.BlockSpec((_EB, d), lambda i: (i, 0)),
            pl.BlockSpec((8, d), lambda i: (0, 0)),
            pl.BlockSpec((1, d), lambda i: (0, 0)),
            pl.BlockSpec((1, d), lambda i: (0, 0)),
        ],
        out_specs=pl.BlockSpec((_EB, d), lambda i: (i, 0)),
        out_shape=jax.ShapeDtypeStruct((e, d), jnp.float32),
        interpret=interpret,
    )(pre, stats, g1, be1)


def _node_mlp(x, sums, cnts, w2aT, w2bT, b2, g2, be2, interpret=False):
    n, d = x.shape
    inv_n = 1.0 / n

    def body(x_ref, su_ref, cn_ref, wa_ref, wb_ref, b_ref, g_ref,
             be_ref, o_ref):
        s = su_ref[0] + su_ref[1]
        cnt = cn_ref[:, 0:1] + cn_ref[:, 1:2]
        agg = s / jnp.maximum(cnt, 1.0)
        p = (jnp.dot(x_ref[...], wa_ref[...],
                     preferred_element_type=jnp.float32)
             + jnp.dot(agg, wb_ref[...], preferred_element_type=jnp.float32)
             + b_ref[...])
        mu = jnp.sum(p, axis=0, keepdims=True) * inv_n
        var = jnp.sum(p * p, axis=0, keepdims=True) * inv_n - mu * mu
        rstd = lax.rsqrt(var + 1e-5)
        h = (p - mu) * rstd * g_ref[...] + be_ref[...]
        o_ref[...] = h * jax.nn.sigmoid(h)

    return pl.pallas_call(
        body,
        out_shape=jax.ShapeDtypeStruct((n, d), jnp.float32),
        interpret=interpret,
    )(x, sums, cnts, w2aT, w2bT, b2, g2, be2)


def kernel(x, edge_index, edge_attr, u, batch, W1, b1, g1, be1, W2, b2, g2,
           be2):
    del u, batch
    n, d = x.shape
    e = edge_index.shape[1]
    row = edge_index[0]
    col = edge_index[1]
    w1aT = jnp.transpose(W1[:, :d])
    w1bT = jnp.transpose(W1[:, d:])
    w2aT = jnp.transpose(W2[:, :d])
    w2bT = jnp.transpose(W2[:, d:])
    b1r = b1.reshape(1, d)
    g1r = g1.reshape(1, d)
    be1r = be1.reshape(1, d)
    b2r = b2.reshape(1, d)
    g2r = g2.reshape(1, d)
    be2r = be2.reshape(1, d)
    zeros_nd = jnp.zeros((n, d), jnp.float32)

    a = _node_matmul(x, w1aT)
    ga = _sc_gather_rows(a, row)
    pre, st = _edge_mlp(ga, edge_attr, w1bT, b1r)
    s = _bn_silu_edges(pre, st, e, g1r, be1r)
    sums, cnts = _sc_scatter_add(s, col, n, zeros_nd)
    cntT = jnp.transpose(cnts.reshape(_NC, n))
    out = _node_mlp(x, sums, cntT, w2aT, w2bT, b2r, g2r, be2r)
    return out


# speedup vs baseline: 1.2524x; 1.0863x over previous
import functools

import jax
import jax.numpy as jnp
from jax import lax
from jax.experimental import pallas as pl
from jax.experimental.pallas import tpu as pltpu
from jax.experimental.pallas import tpu_sc as plsc

_NC = 2
_NS = 16
_NW = _NC * _NS

_D = 128
_EB = 3200
_SC_CHUNK = 80


def _node_matmul(x, wT, interpret=False):
    n, d = x.shape

    def body(x_ref, w_ref, o_ref):
        o_ref[...] = jnp.dot(x_ref[...], w_ref[...],
                             preferred_element_type=jnp.float32)

    return pl.pallas_call(
        body,
        out_shape=jax.ShapeDtypeStruct((n, wT.shape[1]), jnp.float32),
        interpret=interpret,
    )(x, wT)


_BIG = 768
_NSUB = _BIG // 128


def _sc_gather_rows(table, idx):
    n, d = table.shape
    e = idx.shape[0]
    per_w = e // _NW
    steps = per_w // _BIG
    rem = per_w - steps * _BIG
    n128 = rem // 128
    tail = rem - n128 * 128
    mesh = plsc.VectorSubcoreMesh(core_axis_name="c", subcore_axis_name="s")

    @functools.partial(
        pl.kernel,
        out_type=jax.ShapeDtypeStruct((e, d), jnp.float32),
        mesh=mesh,
        scratch_types=[
            pltpu.VMEM((_BIG,), jnp.int32),
            pltpu.VMEM((128,), jnp.int32),
            pltpu.VMEM((_BIG, d), jnp.float32),
            pltpu.SemaphoreType.DMA,
        ],
    )
    def k(table_hbm, idx_hbm, out_hbm, idx_v, idxt_v, rows_v, sem):
        cid = lax.axis_index("c")
        sid = lax.axis_index("s")
        wid = sid * _NC + cid
        base0 = wid * per_w

        @pl.loop(0, steps)
        def _(i):
            base = base0 + i * _BIG
            pltpu.sync_copy(idx_hbm.at[pl.ds(base, _BIG)], idx_v)
            hs = [
                pltpu.async_copy(
                    table_hbm.at[idx_v.at[pl.ds(128 * j, 128)]],
                    rows_v.at[pl.ds(128 * j, 128)], sem)
                for j in range(_NSUB)
            ]
            for h in hs:
                h.wait()
            pltpu.sync_copy(rows_v, out_hbm.at[pl.ds(base, _BIG)])

        rbase = base0 + steps * _BIG
        rem_sizes = [128] * n128 + ([tail] if tail else [])
        off = 0
        for sz in rem_sizes:
            pltpu.sync_copy(idx_hbm.at[pl.ds(rbase + off, sz)],
                            idxt_v.at[pl.ds(0, sz)])
            pltpu.async_copy(table_hbm.at[idxt_v.at[pl.ds(0, sz)]],
                             rows_v.at[pl.ds(0, sz)], sem).wait()
            pltpu.sync_copy(rows_v.at[pl.ds(0, sz)],
                            out_hbm.at[pl.ds(rbase + off, sz)])
            off += sz

    return k(table, idx)


def _sc_scatter_add(vals, col, n, zeros_nd):
    e, d = vals.shape
    per_w = e // _NW
    steps = per_w // 128
    etail = per_w - steps * 128
    rpt = (n // _NS) // 8 * 8
    tail = n - 15 * rpt
    mesh = plsc.VectorSubcoreMesh(core_axis_name="c", subcore_axis_name="s")

    @functools.partial(
        pl.kernel,
        out_type=(jax.ShapeDtypeStruct((_NC, n, d), jnp.float32),
                  jax.ShapeDtypeStruct((_NC * n,), jnp.float32)),
        mesh=mesh,
        scratch_types=(
            [pltpu.VMEM((128,), jnp.int32) for _ in range(2)]
            + [pltpu.VMEM((128, d), jnp.float32) for _ in range(2)]
            + [
                pltpu.VMEM((max(etail, 8),), jnp.int32),
                pltpu.VMEM((128,), jnp.float32),
                pltpu.VMEM((tail,), jnp.float32),
                pltpu.VMEM_SHARED((n, d), jnp.float32),
                pltpu.VMEM_SHARED((n,), jnp.float32),
                pltpu.SemaphoreType.DMA,
                pltpu.SemaphoreType.DMA,
            ]
        ),
    )
    def k(vals_hbm, col_hbm, znd_hbm, sums_hbm, cnts_hbm,
          i0, i1, v0, v1, idxt_v, ones_v, cbuf_v,
          acc_s, cnt_s, sem0, sem1):
        idxb = [i0, i1]
        valsb = [v0, v1]
        semb = [sem0, sem1]
        cid = lax.axis_index("c")
        sid = lax.axis_index("s")
        wid = sid * _NC + cid
        r0 = sid * rpt

        for i in range(tail // 16):
            cbuf_v[pl.ds(16 * i, 16)] = jnp.zeros((16,), jnp.float32)
        for i in range(128 // 16):
            ones_v[pl.ds(16 * i, 16)] = jnp.full((16,), 1.0, jnp.float32)

        @pl.when(sid < _NS - 1)
        def _():
            pltpu.sync_copy(znd_hbm.at[pl.ds(r0, rpt)],
                            acc_s.at[pl.ds(r0, rpt)])
            pltpu.sync_copy(cbuf_v.at[pl.ds(0, rpt)],
                            cnt_s.at[pl.ds(r0, rpt)])

        @pl.when(sid == _NS - 1)
        def _():
            pltpu.sync_copy(znd_hbm.at[pl.ds(15 * rpt, tail)],
                            acc_s.at[pl.ds(15 * rpt, tail)])
            pltpu.sync_copy(cbuf_v, cnt_s.at[pl.ds(15 * rpt, tail)])

        plsc.subcore_barrier()

        base0 = wid * per_w

        for b in (0, 1):
            pltpu.async_copy(col_hbm.at[pl.ds(base0 + 128 * b, 128)],
                             idxb[b], semb[b])
            pltpu.async_copy(vals_hbm.at[pl.ds(base0 + 128 * b, 128)],
                             valsb[b], semb[b])

        @pl.loop(0, steps // 2)
        def _(pit):
            for b in (0, 1):
                k2 = 2 * pit + b
                pltpu.make_async_copy(
                    col_hbm.at[pl.ds(base0, 128)], idxb[b], semb[b]).wait()
                pltpu.make_async_copy(
                    vals_hbm.at[pl.ds(base0, 128)], valsb[b],
                    semb[b]).wait()
                pltpu.sync_copy(valsb[b], acc_s.at[idxb[b]], add=True)
                pltpu.sync_copy(ones_v, cnt_s.at[idxb[b]], add=True)

                @pl.when(k2 + 2 < steps)
                def _():
                    nb = base0 + (k2 + 2) * 128
                    pltpu.async_copy(col_hbm.at[pl.ds(nb, 128)], idxb[b],
                                     semb[b])
                    pltpu.async_copy(vals_hbm.at[pl.ds(nb, 128)], valsb[b],
                                     semb[b])

        if etail:
            rbase = base0 + steps * 128
            pltpu.sync_copy(col_hbm.at[pl.ds(rbase, etail)], idxt_v)
            pltpu.sync_copy(vals_hbm.at[pl.ds(rbase, etail)],
                            v0.at[pl.ds(0, etail)])
            pltpu.sync_copy(v0.at[pl.ds(0, etail)], acc_s.at[idxt_v],
                            add=True)
            pltpu.sync_copy(ones_v.at[pl.ds(0, etail)], cnt_s.at[idxt_v],
                            add=True)

        plsc.subcore_barrier()

        @pl.when(sid < _NS - 1)
        def _():
            pltpu.sync_copy(acc_s.at[pl.ds(r0, rpt)],
                            sums_hbm.at[cid, pl.ds(r0, rpt)])
            pltpu.sync_copy(cnt_s.at[pl.ds(r0, rpt)],
                            cbuf_v.at[pl.ds(0, rpt)])
            pltpu.sync_copy(cbuf_v.at[pl.ds(0, rpt)],
                            cnts_hbm.at[pl.ds(cid * n + r0, rpt)])

        @pl.when(sid == _NS - 1)
        def _():
            pltpu.sync_copy(acc_s.at[pl.ds(15 * rpt, tail)],
                            sums_hbm.at[cid, pl.ds(15 * rpt, tail)])
            pltpu.sync_copy(cnt_s.at[pl.ds(15 * rpt, tail)], cbuf_v)
            pltpu.sync_copy(cbuf_v,
                            cnts_hbm.at[pl.ds(cid * n + 15 * rpt, tail)])

    return k(vals, col, zeros_nd)


def _edge_mlp(g, ea, w1bT, b1, interpret=False):
    e, d = ea.shape
    steps = e // _EB

    def body(g_ref, ea_ref, w_ref, b_ref, pre_ref, st_ref, s1_ref, s2_ref):
        i = pl.program_id(0)

        @pl.when(i == 0)
        def _():
            s1_ref[...] = jnp.zeros_like(s1_ref)
            s2_ref[...] = jnp.zeros_like(s2_ref)

        p = g_ref[...] + jnp.dot(ea_ref[...], w_ref[...],
                                 preferred_element_type=jnp.float32) + b_ref[...]
        pre_ref[...] = p.astype(jnp.bfloat16)
        s1_ref[...] += jnp.sum(p, axis=0, keepdims=True)
        s2_ref[...] += jnp.sum(p * p, axis=0, keepdims=True)
        st_ref[0:1, :] = s1_ref[...]
        st_ref[1:2, :] = s2_ref[...]

    return pl.pallas_call(
        body,
        grid=(steps,),
        in_specs=[
            pl.BlockSpec((_EB, d), lambda i: (i, 0)),
            pl.BlockSpec((_EB, d), lambda i: (i, 0)),
            pl.BlockSpec((d, d), lambda i: (0, 0)),
            pl.BlockSpec((1, d), lambda i: (0, 0)),
        ],
        out_specs=[
            pl.BlockSpec((_EB, d), lambda i: (i, 0)),
            pl.BlockSpec((8, d), lambda i: (0, 0)),
        ],
        out_shape=[
            jax.ShapeDtypeStruct((e, d), jnp.bfloat16),
            jax.ShapeDtypeStruct((8, d), jnp.float32),
        ],
        scratch_shapes=[
            pltpu.VMEM((1, d), jnp.float32),
            pltpu.VMEM((1, d), jnp.float32),
        ],
        interpret=interpret,
    )(g, ea, w1bT, b1)


def _bn_silu_edges(pre, stats, e_total, g1, be1, interpret=False):
    e, d = pre.shape
    steps = e // _EB
    inv_e = 1.0 / e_total

    def body(pre_ref, st_ref, g_ref, b_ref, s_ref):
        st = st_ref[...]
        mu = st[0:1, :] * inv_e
        var = st[1:2, :] * inv_e - mu * mu
        rstd = lax.rsqrt(var + 1e-5)
        w = g_ref[...] * rstd
        cc = b_ref[...] - mu * w
        h = pre_ref[...].astype(jnp.float32) * w + cc
        s_ref[...] = h * jax.nn.sigmoid(h)

    return pl.pallas_call(
        body,
        grid=(steps,),
        in_specs=[
            pl.BlockSpec((_EB, d), lambda i: (i, 0)),
            pl.BlockSpec((8, d), lambda i: (0, 0)),
            pl.BlockSpec((1, d), lambda i: (0, 0)),
            pl.BlockSpec((1, d), lambda i: (0, 0)),
        ],
        out_specs=pl.BlockSpec((_EB, d), lambda i: (i, 0)),
        out_shape=jax.ShapeDtypeStruct((e, d), jnp.float32),
        interpret=interpret,
    )(pre, stats, g1, be1)


def _node_mlp(x, sums, cnts, w2aT, w2bT, b2, g2, be2, interpret=False):
    n, d = x.shape
    inv_n = 1.0 / n

    def body(x_ref, su_ref, cn_ref, wa_ref, wb_ref, b_ref, g_ref,
             be_ref, o_ref):
        s = su_ref[0] + su_ref[1]
        cnt = cn_ref[:, 0:1] + cn_ref[:, 1:2]
        agg = s / jnp.maximum(cnt, 1.0)
        p = (jnp.dot(x_ref[...], wa_ref[...],
                     preferred_element_type=jnp.float32)
             + jnp.dot(agg, wb_ref[...], preferred_element_type=jnp.float32)
             + b_ref[...])
        mu = jnp.sum(p, axis=0, keepdims=True) * inv_n
        var = jnp.sum(p * p, axis=0, keepdims=True) * inv_n - mu * mu
        rstd = lax.rsqrt(var + 1e-5)
        h = (p - mu) * rstd * g_ref[...] + be_ref[...]
        o_ref[...] = h * jax.nn.sigmoid(h)

    return pl.pallas_call(
        body,
        out_shape=jax.ShapeDtypeStruct((n, d), jnp.float32),
        interpret=interpret,
    )(x, sums, cnts, w2aT, w2bT, b2, g2, be2)


def kernel(x, edge_index, edge_attr, u, batch, W1, b1, g1, be1, W2, b2, g2,
           be2):
    del u, batch
    n, d = x.shape
    e = edge_index.shape[1]
    row = edge_index[0]
    col = edge_index[1]
    w1aT = jnp.transpose(W1[:, :d])
    w1bT = jnp.transpose(W1[:, d:])
    w2aT = jnp.transpose(W2[:, :d])
    w2bT = jnp.transpose(W2[:, d:])
    b1r = b1.reshape(1, d)
    g1r = g1.reshape(1, d)
    be1r = be1.reshape(1, d)
    b2r = b2.reshape(1, d)
    g2r = g2.reshape(1, d)
    be2r = be2.reshape(1, d)
    zeros_nd = jnp.zeros((n, d), jnp.float32)

    a = _node_matmul(x, w1aT)
    ga = _sc_gather_rows(a, row)
    pre, st = _edge_mlp(ga, edge_attr, w1bT, b1r)
    s = _bn_silu_edges(pre, st, e, g1r, be1r)
    sums, cnts = _sc_scatter_add(s, col, n, zeros_nd)
    cntT = jnp.transpose(cnts.reshape(_NC, n))
    out = _node_mlp(x, sums, cntT, w2aT, w2bT, b2r, g2r, be2r)
    return out


# speedup vs baseline: 1.2606x; 1.0066x over previous
import functools

import jax
import jax.numpy as jnp
from jax import lax
from jax.experimental import pallas as pl
from jax.experimental.pallas import tpu as pltpu
from jax.experimental.pallas import tpu_sc as plsc

_NC = 2
_NS = 16
_NW = _NC * _NS

_D = 128
_EB = 3200
_SC_CHUNK = 80


def _node_matmul(x, wT, interpret=False):
    n, d = x.shape

    def body(x_ref, w_ref, o_ref):
        o_ref[...] = jnp.dot(x_ref[...], w_ref[...],
                             preferred_element_type=jnp.float32)

    return pl.pallas_call(
        body,
        out_shape=jax.ShapeDtypeStruct((n, wT.shape[1]), jnp.float32),
        interpret=interpret,
    )(x, wT)


_BIG = 768
_NSUB = _BIG // 128


def _sc_gather_rows(table, idx):
    n, d = table.shape
    e = idx.shape[0]
    big = 384
    nsub = big // 128
    per_w = e // _NW
    steps = per_w // big
    rem = per_w - steps * big
    n128 = rem // 128
    tail = rem - n128 * 128
    mesh = plsc.VectorSubcoreMesh(core_axis_name="c", subcore_axis_name="s")

    @functools.partial(
        pl.kernel,
        out_type=jax.ShapeDtypeStruct((e, d), jnp.float32),
        mesh=mesh,
        scratch_types=[
            pltpu.VMEM((big,), jnp.int32),
            pltpu.VMEM((big,), jnp.int32),
            pltpu.VMEM((128,), jnp.int32),
            pltpu.VMEM((big, d), jnp.float32),
            pltpu.VMEM((big, d), jnp.float32),
            pltpu.SemaphoreType.DMA,
            pltpu.SemaphoreType.DMA,
            pltpu.SemaphoreType.DMA,
            pltpu.SemaphoreType.DMA,
        ],
    )
    def k(table_hbm, idx_hbm, out_hbm, ia, ib, idxt_v, ra, rb,
          semi0, semi1, semw0, semw1):
        idxv = [ia, ib]
        rowsv = [ra, rb]
        semi = [semi0, semi1]
        semw = [semw0, semw1]
        cid = lax.axis_index("c")
        sid = lax.axis_index("s")
        wid = sid * _NC + cid
        base0 = wid * per_w

        for b in (0, 1):
            pltpu.async_copy(idx_hbm.at[pl.ds(base0 + big * b, big)],
                             idxv[b], semi[b])

        @pl.loop(0, steps // 2)
        def _(pit):
            for b in (0, 1):
                k2 = 2 * pit + b
                pltpu.make_async_copy(idx_hbm.at[pl.ds(base0, big)],
                                      idxv[b], semi[b]).wait()

                @pl.when(k2 >= 2)
                def _():
                    pltpu.make_async_copy(
                        rowsv[b], out_hbm.at[pl.ds(base0, big)],
                        semw[b]).wait()

                hs = [
                    pltpu.async_copy(
                        table_hbm.at[idxv[b].at[pl.ds(128 * j, 128)]],
                        rowsv[b].at[pl.ds(128 * j, 128)], semi[b])
                    for j in range(nsub)
                ]
                for h in hs:
                    h.wait()
                base = base0 + k2 * big
                pltpu.async_copy(rowsv[b], out_hbm.at[pl.ds(base, big)],
                                 semw[b])

                @pl.when(k2 + 2 < steps)
                def _():
                    nb = base0 + (k2 + 2) * big
                    pltpu.async_copy(idx_hbm.at[pl.ds(nb, big)], idxv[b],
                                     semi[b])

        for b in (0, 1):
            pltpu.make_async_copy(rowsv[b], out_hbm.at[pl.ds(base0, big)],
                                  semw[b]).wait()

        rbase = base0 + steps * big
        rem_sizes = [128] * n128 + ([tail] if tail else [])
        off = 0
        for sz in rem_sizes:
            pltpu.sync_copy(idx_hbm.at[pl.ds(rbase + off, sz)],
                            idxt_v.at[pl.ds(0, sz)])
            pltpu.async_copy(table_hbm.at[idxt_v.at[pl.ds(0, sz)]],
                             ra.at[pl.ds(0, sz)], semi0).wait()
            pltpu.sync_copy(ra.at[pl.ds(0, sz)],
                            out_hbm.at[pl.ds(rbase + off, sz)])
            off += sz

    return k(table, idx)


def _sc_scatter_add(vals, col, n, zeros_nd):
    e, d = vals.shape
    per_w = e // _NW
    steps = per_w // 128
    etail = per_w - steps * 128
    rpt = (n // _NS) // 8 * 8
    tail = n - 15 * rpt
    mesh = plsc.VectorSubcoreMesh(core_axis_name="c", subcore_axis_name="s")

    @functools.partial(
        pl.kernel,
        out_type=(jax.ShapeDtypeStruct((_NC, n, d), jnp.float32),
                  jax.ShapeDtypeStruct((_NC * n,), jnp.float32)),
        mesh=mesh,
        scratch_types=(
            [pltpu.VMEM((128,), jnp.int32) for _ in range(2)]
            + [pltpu.VMEM((128, d), jnp.float32) for _ in range(2)]
            + [
                pltpu.VMEM((max(etail, 8),), jnp.int32),
                pltpu.VMEM((128,), jnp.float32),
                pltpu.VMEM((tail,), jnp.float32),
                pltpu.VMEM_SHARED((n, d), jnp.float32),
                pltpu.VMEM_SHARED((n,), jnp.float32),
                pltpu.SemaphoreType.DMA,
                pltpu.SemaphoreType.DMA,
            ]
        ),
    )
    def k(vals_hbm, col_hbm, znd_hbm, sums_hbm, cnts_hbm,
          i0, i1, v0, v1, idxt_v, ones_v, cbuf_v,
          acc_s, cnt_s, sem0, sem1):
        idxb = [i0, i1]
        valsb = [v0, v1]
        semb = [sem0, sem1]
        cid = lax.axis_index("c")
        sid = lax.axis_index("s")
        wid = sid * _NC + cid
        r0 = sid * rpt

        for i in range(tail // 16):
            cbuf_v[pl.ds(16 * i, 16)] = jnp.zeros((16,), jnp.float32)
        for i in range(128 // 16):
            ones_v[pl.ds(16 * i, 16)] = jnp.full((16,), 1.0, jnp.float32)

        @pl.when(sid < _NS - 1)
        def _():
            pltpu.sync_copy(znd_hbm.at[pl.ds(r0, rpt)],
                            acc_s.at[pl.ds(r0, rpt)])
            pltpu.sync_copy(cbuf_v.at[pl.ds(0, rpt)],
                            cnt_s.at[pl.ds(r0, rpt)])

        @pl.when(sid == _NS - 1)
        def _():
            pltpu.sync_copy(znd_hbm.at[pl.ds(15 * rpt, tail)],
                            acc_s.at[pl.ds(15 * rpt, tail)])
            pltpu.sync_copy(cbuf_v, cnt_s.at[pl.ds(15 * rpt, tail)])

        plsc.subcore_barrier()

        base0 = wid * per_w

        for b in (0, 1):
            pltpu.async_copy(col_hbm.at[pl.ds(base0 + 128 * b, 128)],
                             idxb[b], semb[b])
            pltpu.async_copy(vals_hbm.at[pl.ds(base0 + 128 * b, 128)],
                             valsb[b], semb[b])

        @pl.loop(0, steps // 2)
        def _(pit):
            for b in (0, 1):
                k2 = 2 * pit + b
                pltpu.make_async_copy(
                    col_hbm.at[pl.ds(base0, 128)], idxb[b], semb[b]).wait()
                pltpu.make_async_copy(
                    vals_hbm.at[pl.ds(base0, 128)], valsb[b],
                    semb[b]).wait()
                pltpu.sync_copy(valsb[b], acc_s.at[idxb[b]], add=True)
                pltpu.sync_copy(ones_v, cnt_s.at[idxb[b]], add=True)

                @pl.when(k2 + 2 < steps)
                def _():
                    nb = base0 + (k2 + 2) * 128
                    pltpu.async_copy(col_hbm.at[pl.ds(nb, 128)], idxb[b],
                                     semb[b])
                    pltpu.async_copy(vals_hbm.at[pl.ds(nb, 128)], valsb[b],
                                     semb[b])

        if etail:
            rbase = base0 + steps * 128
            pltpu.sync_copy(col_hbm.at[pl.ds(rbase, etail)], idxt_v)
            pltpu.sync_copy(vals_hbm.at[pl.ds(rbase, etail)],
                            v0.at[pl.ds(0, etail)])
            pltpu.sync_copy(v0.at[pl.ds(0, etail)], acc_s.at[idxt_v],
                            add=True)
            pltpu.sync_copy(ones_v.at[pl.ds(0, etail)], cnt_s.at[idxt_v],
                            add=True)

        plsc.subcore_barrier()

        @pl.when(sid < _NS - 1)
        def _():
            pltpu.sync_copy(acc_s.at[pl.ds(r0, rpt)],
                            sums_hbm.at[cid, pl.ds(r0, rpt)])
            pltpu.sync_copy(cnt_s.at[pl.ds(r0, rpt)],
                            cbuf_v.at[pl.ds(0, rpt)])
            pltpu.sync_copy(cbuf_v.at[pl.ds(0, rpt)],
                            cnts_hbm.at[pl.ds(cid * n + r0, rpt)])

        @pl.when(sid == _NS - 1)
        def _():
            pltpu.sync_copy(acc_s.at[pl.ds(15 * rpt, tail)],
                            sums_hbm.at[cid, pl.ds(15 * rpt, tail)])
            pltpu.sync_copy(cnt_s.at[pl.ds(15 * rpt, tail)], cbuf_v)
            pltpu.sync_copy(cbuf_v,
                            cnts_hbm.at[pl.ds(cid * n + 15 * rpt, tail)])

    return k(vals, col, zeros_nd)


def _edge_mlp(g, ea, w1bT, b1, interpret=False):
    e, d = ea.shape
    steps = e // _EB

    def body(g_ref, ea_ref, w_ref, b_ref, pre_ref, st_ref, s1_ref, s2_ref):
        i = pl.program_id(0)

        @pl.when(i == 0)
        def _():
            s1_ref[...] = jnp.zeros_like(s1_ref)
            s2_ref[...] = jnp.zeros_like(s2_ref)

        p = g_ref[...] + jnp.dot(ea_ref[...], w_ref[...],
                                 preferred_element_type=jnp.float32) + b_ref[...]
        pre_ref[...] = p.astype(jnp.bfloat16)
        s1_ref[...] += jnp.sum(p, axis=0, keepdims=True)
        s2_ref[...] += jnp.sum(p * p, axis=0, keepdims=True)
        st_ref[0:1, :] = s1_ref[...]
        st_ref[1:2, :] = s2_ref[...]

    return pl.pallas_call(
        body,
        grid=(steps,),
        in_specs=[
            pl.BlockSpec((_EB, d), lambda i: (i, 0)),
            pl.BlockSpec((_EB, d), lambda i: (i, 0)),
            pl.BlockSpec((d, d), lambda i: (0, 0)),
            pl.BlockSpec((1, d), lambda i: (0, 0)),
        ],
        out_specs=[
            pl.BlockSpec((_EB, d), lambda i: (i, 0)),
            pl.BlockSpec((8, d), lambda i: (0, 0)),
        ],
        out_shape=[
            jax.ShapeDtypeStruct((e, d), jnp.bfloat16),
            jax.ShapeDtypeStruct((8, d), jnp.float32),
        ],
        scratch_shapes=[
            pltpu.VMEM((1, d), jnp.float32),
            pltpu.VMEM((1, d), jnp.float32),
        ],
        interpret=interpret,
    )(g, ea, w1bT, b1)


def _bn_silu_edges(pre, stats, e_total, g1, be1, interpret=False):
    e, d = pre.shape
    steps = e // _EB
    inv_e = 1.0 / e_total

    def body(pre_ref, st_ref, g_ref, b_ref, s_ref):
        st = st_ref[...]
        mu = st[0:1, :] * inv_e
        var = st[1:2, :] * inv_e - mu * mu
        rstd = lax.rsqrt(var + 1e-5)
        w = g_ref[...] * rstd
        cc = b_ref[...] - mu * w
        h = pre_ref[...].astype(jnp.float32) * w + cc
        s_ref[...] = h * jax.nn.sigmoid(h)

    return pl.pallas_call(
        body,
        grid=(steps,),
        in_specs=[
            pl.BlockSpec((_EB, d), lambda i: (i, 0)),
            pl.BlockSpec((8, d), lambda i: (0, 0)),
            pl.BlockSpec((1, d), lambda i: (0, 0)),
            pl.BlockSpec((1, d), lambda i: (0, 0)),
        ],
        out_specs=pl.BlockSpec((_EB, d), lambda i: (i, 0)),
        out_shape=jax.ShapeDtypeStruct((e, d), jnp.float32),
        interpret=interpret,
    )(pre, stats, g1, be1)


def _node_mlp(x, sums, cnts, w2aT, w2bT, b2, g2, be2, interpret=False):
    n, d = x.shape
    inv_n = 1.0 / n

    def body(x_ref, su_ref, cn_ref, wa_ref, wb_ref, b_ref, g_ref,
             be_ref, o_ref):
        s = su_ref[0] + su_ref[1]
        cnt = cn_ref[:, 0:1] + cn_ref[:, 1:2]
        agg = s / jnp.maximum(cnt, 1.0)
        p = (jnp.dot(x_ref[...], wa_ref[...],
                     preferred_element_type=jnp.float32)
             + jnp.dot(agg, wb_ref[...], preferred_element_type=jnp.float32)
             + b_ref[...])
        mu = jnp.sum(p, axis=0, keepdims=True) * inv_n
        var = jnp.sum(p * p, axis=0, keepdims=True) * inv_n - mu * mu
        rstd = lax.rsqrt(var + 1e-5)
        h = (p - mu) * rstd * g_ref[...] + be_ref[...]
        o_ref[...] = h * jax.nn.sigmoid(h)

    return pl.pallas_call(
        body,
        out_shape=jax.ShapeDtypeStruct((n, d), jnp.float32),
        interpret=interpret,
    )(x, sums, cnts, w2aT, w2bT, b2, g2, be2)


def kernel(x, edge_index, edge_attr, u, batch, W1, b1, g1, be1, W2, b2, g2,
           be2):
    del u, batch
    n, d = x.shape
    e = edge_index.shape[1]
    row = edge_index[0]
    col = edge_index[1]
    w1aT = jnp.transpose(W1[:, :d])
    w1bT = jnp.transpose(W1[:, d:])
    w2aT = jnp.transpose(W2[:, :d])
    w2bT = jnp.transpose(W2[:, d:])
    b1r = b1.reshape(1, d)
    g1r = g1.reshape(1, d)
    be1r = be1.reshape(1, d)
    b2r = b2.reshape(1, d)
    g2r = g2.reshape(1, d)
    be2r = be2.reshape(1, d)
    zeros_nd = jnp.zeros((n, d), jnp.float32)

    a = _node_matmul(x, w1aT)
    ga = _sc_gather_rows(a, row)
    pre, st = _edge_mlp(ga, edge_attr, w1bT, b1r)
    s = _bn_silu_edges(pre, st, e, g1r, be1r)
    sums, cnts = _sc_scatter_add(s, col, n, zeros_nd)
    cntT = jnp.transpose(cnts.reshape(_NC, n))
    out = _node_mlp(x, sums, cntT, w2aT, w2bT, b2r, g2r, be2r)
    return out
